# baseline (speedup 1.0000x reference)
import functools

import jax
import jax.numpy as jnp
from jax.experimental import pallas as pl
from jax.experimental.pallas import tpu as pltpu

_NBUF = 4
_K = 1


def _se_attention(x, w1t, b1, w2t, b2, inv_hw):
    s = jnp.sum(x, axis=-1) * inv_hw + jnp.max(x, axis=-1)
    h = jnp.dot(s, w1t, preferred_element_type=jnp.float32)
    h = jnp.maximum(h + b1, 0.0)
    a = jnp.dot(h, w2t, preferred_element_type=jnp.float32)
    att = 1.0 + jax.nn.sigmoid(a + b2)
    return x * att[:, :, None]


def _se_kernel(x_hbm, w1t_ref, b1_ref, w2t_ref, b2_ref, o_hbm,
               ibufs, obufs, isems, osems, *, inv_hw, half):
    i = pl.program_id(0)
    n = pl.num_programs(0)

    def rd(step, slot, s):
        return pltpu.make_async_copy(
            x_hbm.at[pl.ds(s * half + step * _K, _K)],
            ibufs.at[slot, s],
            isems.at[slot, s],
        )

    def wr(step, slot, s):
        return pltpu.make_async_copy(
            obufs.at[slot, s],
            o_hbm.at[pl.ds(s * half + step * _K, _K)],
            osems.at[slot, s],
        )

    @pl.when(i == 0)
    def _():
        for j in range(min(_NBUF, n)):
            rd(j, j, 0).start()
            rd(j, j, 1).start()

    slot = jax.lax.rem(i, _NBUF)
    rd(i, slot, 0).wait()
    rd(i, slot, 1).wait()

    @pl.when(i >= _NBUF)
    def _():
        wr(i - _NBUF, slot, 0).wait()
        wr(i - _NBUF, slot, 1).wait()

    w1t = w1t_ref[...]
    b1 = b1_ref[...]
    w2t = w2t_ref[...]
    b2 = b2_ref[...]
    obufs[slot, 0] = _se_attention(ibufs[slot, 0], w1t, b1, w2t, b2, inv_hw)
    obufs[slot, 1] = _se_attention(ibufs[slot, 1], w1t, b1, w2t, b2, inv_hw)

    wr(i, slot, 0).start()
    wr(i, slot, 1).start()

    @pl.when(i + _NBUF < n)
    def _():
        rd(i + _NBUF, slot, 0).start()
        rd(i + _NBUF, slot, 1).start()

    @pl.when(i == n - 1)
    def _():
        for j in range(min(_NBUF, n)):
            step = n - min(_NBUF, n) + j
            wr(step, step % _NBUF, 0).wait()
            wr(step, step % _NBUF, 1).wait()


def kernel(x, w1, b1, w2, b2):
    B, C, H, W = x.shape
    Cr = w1.shape[0]
    HW = H * W
    inv_hw = 1.0 / HW
    half = B // 2
    n = half // _K

    x_k = x.reshape(B, C, HW)
    w1t = jnp.transpose(w1)
    b1_2d = b1.reshape(1, Cr)
    w2t = jnp.transpose(w2)
    b2_2d = b2.reshape(1, C)

    out_k = pl.pallas_call(
        functools.partial(_se_kernel, inv_hw=inv_hw, half=half),
        out_shape=jax.ShapeDtypeStruct((B, C, HW), x.dtype),
        grid=(n,),
        in_specs=[
            pl.BlockSpec(memory_space=pl.ANY),
            pl.BlockSpec((C, Cr), lambda i: (0, 0)),
            pl.BlockSpec((1, Cr), lambda i: (0, 0)),
            pl.BlockSpec((Cr, C), lambda i: (0, 0)),
            pl.BlockSpec((1, C), lambda i: (0, 0)),
        ],
        out_specs=pl.BlockSpec(memory_space=pl.ANY),
        scratch_shapes=[
            pltpu.VMEM((_NBUF, 2, _K, C, HW), jnp.float32),
            pltpu.VMEM((_NBUF, 2, _K, C, HW), jnp.float32),
            pltpu.SemaphoreType.DMA((_NBUF, 2)),
            pltpu.SemaphoreType.DMA((_NBUF, 2)),
        ],
        compiler_params=pltpu.CompilerParams(
            dimension_semantics=("arbitrary",),
            vmem_limit_bytes=60 << 20,
        ),
        cost_estimate=pl.CostEstimate(
            flops=int(4 * B * C * HW + 4 * B * C * Cr),
            transcendentals=int(B * C),
            bytes_accessed=int(2 * B * C * HW * 4),
        ),
    )(x_k, w1t, b1_2d, w2t, b2_2d)
    return out_k.reshape(B, C, H, W)
